# baseline (speedup 1.0000x reference)
import jax
import jax.numpy as jnp
from jax import lax
from jax.experimental import pallas as pl
from jax.experimental.pallas import tpu as pltpu

_BETA_THRESHOLD = 0.3
_DIST_THRESHOLD = 0.5
_N = 20000
_ROWS = 160
_COLS = 128
_NPAD = _ROWS * _COLS

_NEG_BIG = -3.0e38


def _condense_kernel(cx_ref, cy_ref, cz_ref, beta_ref, dist_ref,
                     assign_ref, alpha_ref, avail_ref):
    flat = (lax.broadcasted_iota(jnp.int32, (_ROWS, _COLS), 0) * _COLS
            + lax.broadcasted_iota(jnp.int32, (_ROWS, _COLS), 1))
    lane = lax.broadcasted_iota(jnp.int32, (1, _COLS), 1)

    assign_ref[:] = jnp.full((_ROWS, _COLS), -1, jnp.int32)
    beta = beta_ref[:]
    avail_ref[:] = beta

    def argmax_avail(avail):
        m = jnp.max(avail)
        a = jnp.min(jnp.where(avail == m, flat, jnp.int32(2**30)))
        return m, a

    m0, a0 = argmax_avail(beta)

    def extract(ref, row, lanemask):
        rowvec = ref[pl.ds(row, 1), :]
        return jnp.max(jnp.where(lanemask, rowvec, _NEG_BIG))

    def body(state):
        k, a, _m = state
        row = a // _COLS
        lanemask = lane == (a % _COLS)
        ax = extract(cx_ref, row, lanemask)
        ay = extract(cy_ref, row, lanemask)
        az = extract(cz_ref, row, lanemask)
        ra = extract(dist_ref, row, lanemask) * jnp.float32(_DIST_THRESHOLD)

        dx = cx_ref[:] - ax
        dy = cy_ref[:] - ay
        dz = cz_ref[:] - az
        d = jnp.sqrt(dx * dx + dy * dy + dz * dz)
        inrad = d <= ra
        avail = avail_ref[:]
        within = inrad & (avail >= 0.0)
        pk = assign_ref[:]
        assign_ref[:] = jnp.where(within, k * jnp.int32(32768) + a, pk)
        avail2 = jnp.where(inrad, jnp.float32(-1.0), avail)
        avail_ref[:] = avail2

        m2, a2 = argmax_avail(avail2)
        return k + jnp.int32(1), a2, m2

    lax.while_loop(lambda s: s[2] > jnp.float32(_BETA_THRESHOLD), body,
                   (jnp.int32(0), a0, m0))

    pk = assign_ref[:]
    unassigned = pk < 0
    assign_ref[:] = jnp.where(unassigned, -1, pk // jnp.int32(32768))
    alpha_ref[:] = jnp.where(unassigned, -1,
                             pk - (pk // jnp.int32(32768)) * jnp.int32(32768))


def kernel(pred_ccoords, pred_beta, pred_dist):
    pad = _NPAD - _N
    cx = jnp.pad(pred_ccoords[:, 0], (0, pad), constant_values=1e30)
    cy = jnp.pad(pred_ccoords[:, 1], (0, pad), constant_values=1e30)
    cz = jnp.pad(pred_ccoords[:, 2], (0, pad), constant_values=1e30)
    beta = jnp.pad(pred_beta.reshape(-1), (0, pad), constant_values=-1.0)
    dist = jnp.pad(pred_dist.reshape(-1), (0, pad), constant_values=0.0)

    shape2d = (_ROWS, _COLS)
    args = [a.reshape(shape2d) for a in (cx, cy, cz, beta, dist)]

    out_shape = [
        jax.ShapeDtypeStruct(shape2d, jnp.int32),
        jax.ShapeDtypeStruct(shape2d, jnp.int32),
    ]
    assign2d, alpha2d = pl.pallas_call(
        _condense_kernel,
        out_shape=out_shape,
        scratch_shapes=[pltpu.VMEM(shape2d, jnp.float32)],
    )(*args)

    assign = assign2d.reshape(-1)[:_N]
    alpha_idx = alpha2d.reshape(-1)[:_N]
    safe_alpha = jnp.where(alpha_idx < 0, 0, alpha_idx)
    cond_coords = jnp.take(pred_ccoords, safe_alpha, axis=0)
    cond_coords = jnp.where((alpha_idx >= 0)[:, None], cond_coords,
                            jnp.zeros_like(cond_coords))
    return assign, alpha_idx, cond_coords
